# baseline (speedup 1.0000x reference)
import functools

import numpy as np
import jax
import jax.numpy as jnp
from jax import lax
from jax.experimental import pallas as pl
from jax.experimental.pallas import tpu as pltpu
from jax.experimental.pallas import tpu_sc as plsc

N = 10000
E = 160000
NG = 64
H = 32
FEAT = 16
EDGE_FEAT = 4
EDGE_HID = 128
LAYERS = 3
STEPS = 3

NP_ = 10240
EC = 128
NW = 32
CPW = 40
NCHUNK = NW * CPW
EP = NCHUNK * EC
SUP = 8
NSUP = CPW // SUP
TRASH = NP_ - 1
ROWS_PER_TILE = NP_ // 16

_EXP = np.kron(np.eye(H, dtype=np.float32), np.ones((1, H), dtype=np.float32))
_RED = np.tile(np.eye(H, dtype=np.float32), (4, 1))


def _sc_mesh():
    return plsc.VectorSubcoreMesh(core_axis_name="c", subcore_axis_name="s",
                                  num_cores=2, num_subcores=16)


_SC_PARAMS = pltpu.CompilerParams(use_tc_tiling_on_sc=False)


def _sc_gather(h, src2d):
    @functools.partial(
        pl.kernel,
        out_type=jax.ShapeDtypeStruct((EP, H), jnp.float32),
        mesh=_sc_mesh(),
        scratch_types=[
            pltpu.VMEM((CPW, EC), jnp.int32),
            pltpu.VMEM((2, SUP * EC, H), jnp.float32),
            pltpu.SemaphoreType.DMA,
            pltpu.SemaphoreType.DMA,
        ],
        compiler_params=_SC_PARAMS,
    )
    def k(h_hbm, src_hbm, out_hbm, idxs, rows, gsem, ssem):
        wid = lax.axis_index("s") * 2 + lax.axis_index("c")
        base = wid * CPW
        pltpu.sync_copy(src_hbm.at[pl.ds(base, CPW)], idxs)
        stores = [None, None]
        for sc in range(NSUP):
            b = sc % 2
            if stores[b] is not None:
                stores[b].wait()
            descs = []
            for jj in range(SUP):
                descs.append(pltpu.async_copy(
                    h_hbm.at[idxs.at[sc * SUP + jj]],
                    rows.at[b, pl.ds(jj * EC, EC)], gsem))
            for d in descs:
                d.wait()
            st = pltpu.async_copy(
                rows.at[b],
                out_hbm.at[pl.ds((base + sc * SUP) * EC, SUP * EC)], ssem)
            stores[b] = st
        for st in stores:
            if st is not None:
                st.wait()

    return k(h, src2d)


def _sc_gather_deg(h, src2d, dst2d, zeros, ones_rows):
    @functools.partial(
        pl.kernel,
        out_type=(jax.ShapeDtypeStruct((EP, H), jnp.float32),
                  jax.ShapeDtypeStruct((2, NP_, H), jnp.float32)),
        mesh=_sc_mesh(),
        scratch_types=[
            pltpu.VMEM((CPW, EC), jnp.int32),
            pltpu.VMEM((CPW, EC), jnp.int32),
            pltpu.VMEM((2, SUP * EC, H), jnp.float32),
            pltpu.VMEM((EC, H), jnp.float32),
            pltpu.VMEM_SHARED((NP_, H), jnp.float32),
            pltpu.SemaphoreType.DMA,
            pltpu.SemaphoreType.DMA,
            pltpu.SemaphoreType.DMA,
        ],
        compiler_params=_SC_PARAMS,
    )
    def k(h_hbm, src_hbm, dst_hbm, zeros_hbm, ones_hbm, out_hbm, deg_hbm,
          idxs, didxs, rows, ones_v, agg_sh, gsem, ssem, asem):
        c = lax.axis_index("c")
        s_ = lax.axis_index("s")
        wid = s_ * 2 + c
        base = wid * CPW
        dbase = (c * 16 + s_) * CPW

        @pl.when(s_ == 0)
        def _():
            pltpu.sync_copy(zeros_hbm, agg_sh)

        pltpu.sync_copy(src_hbm.at[pl.ds(base, CPW)], idxs)
        pltpu.sync_copy(dst_hbm.at[pl.ds(dbase, CPW)], didxs)
        pltpu.sync_copy(ones_hbm, ones_v)
        plsc.subcore_barrier()

        adds = []
        for j in range(CPW):
            adds.append(pltpu.async_copy(
                ones_v, agg_sh.at[didxs.at[j]], asem, add=True))

        stores = [None, None]
        for sc in range(NSUP):
            b = sc % 2
            if stores[b] is not None:
                stores[b].wait()
            descs = []
            for jj in range(SUP):
                descs.append(pltpu.async_copy(
                    h_hbm.at[idxs.at[sc * SUP + jj]],
                    rows.at[b, pl.ds(jj * EC, EC)], gsem))
            for d in descs:
                d.wait()
            st = pltpu.async_copy(
                rows.at[b],
                out_hbm.at[pl.ds((base + sc * SUP) * EC, SUP * EC)], ssem)
            stores[b] = st
        for st in stores:
            if st is not None:
                st.wait()
        for d in adds:
            d.wait()
        plsc.subcore_barrier()
        pltpu.sync_copy(agg_sh.at[pl.ds(s_ * ROWS_PER_TILE, ROWS_PER_TILE)],
                        deg_hbm.at[c, pl.ds(s_ * ROWS_PER_TILE, ROWS_PER_TILE)])

    return k(h, src2d, dst2d, zeros, ones_rows)


def _sc_scatter(msgs, dst2d, zeros):
    @functools.partial(
        pl.kernel,
        out_type=jax.ShapeDtypeStruct((2, NP_, H), jnp.float32),
        mesh=_sc_mesh(),
        scratch_types=[
            pltpu.VMEM((CPW, EC), jnp.int32),
            pltpu.VMEM((2, SUP * EC, H), jnp.float32),
            pltpu.VMEM_SHARED((NP_, H), jnp.float32),
            pltpu.SemaphoreType.DMA,
            pltpu.SemaphoreType.DMA,
        ],
        compiler_params=_SC_PARAMS,
    )
    def k(msgs_hbm, dst_hbm, zeros_hbm, out_hbm, idxs, rows, agg_sh,
          lsem, asem):
        c = lax.axis_index("c")
        s_ = lax.axis_index("s")
        base = (c * 16 + s_) * CPW

        @pl.when(s_ == 0)
        def _():
            pltpu.sync_copy(zeros_hbm, agg_sh)

        pltpu.sync_copy(dst_hbm.at[pl.ds(base, CPW)], idxs)
        plsc.subcore_barrier()

        loads = [None, None]
        adds = [None] * NSUP
        loads[0] = pltpu.async_copy(
            msgs_hbm.at[pl.ds(base * EC, SUP * EC)], rows.at[0], lsem)
        for sc in range(NSUP):
            b = sc % 2
            loads[b].wait()
            if sc + 1 < NSUP:
                if adds[sc - 1] is not None:
                    for d in adds[sc - 1]:
                        d.wait()
                    adds[sc - 1] = None
                loads[(sc + 1) % 2] = pltpu.async_copy(
                    msgs_hbm.at[pl.ds((base + (sc + 1) * SUP) * EC, SUP * EC)],
                    rows.at[(sc + 1) % 2], lsem)
            batch = []
            for jj in range(SUP):
                batch.append(pltpu.async_copy(
                    rows.at[b, pl.ds(jj * EC, EC)],
                    agg_sh.at[idxs.at[sc * SUP + jj]], asem, add=True))
            adds[sc] = batch
        for batch in adds:
            if batch is not None:
                for d in batch:
                    d.wait()
        plsc.subcore_barrier()
        pltpu.sync_copy(agg_sh.at[pl.ds(s_ * ROWS_PER_TILE, ROWS_PER_TILE)],
                        out_hbm.at[c, pl.ds(s_ * ROWS_PER_TILE, ROWS_PER_TILE)])

    return k(msgs, dst2d, zeros)


def _sc_degree(dst2d, zeros, ones_rows):
    @functools.partial(
        pl.kernel,
        out_type=jax.ShapeDtypeStruct((2, NP_, H), jnp.float32),
        mesh=_sc_mesh(),
        scratch_types=[
            pltpu.VMEM((CPW, EC), jnp.int32),
            pltpu.VMEM((EC, H), jnp.float32),
            pltpu.VMEM_SHARED((NP_, H), jnp.float32),
            pltpu.SemaphoreType.DMA,
        ],
        compiler_params=_SC_PARAMS,
    )
    def k(dst_hbm, zeros_hbm, ones_hbm, out_hbm, idxs, rows_v, agg_sh, asem):
        c = lax.axis_index("c")
        s_ = lax.axis_index("s")
        base = (c * 16 + s_) * CPW

        @pl.when(s_ == 0)
        def _():
            pltpu.sync_copy(zeros_hbm, agg_sh)

        pltpu.sync_copy(dst_hbm.at[pl.ds(base, CPW)], idxs)
        pltpu.sync_copy(ones_hbm, rows_v)
        plsc.subcore_barrier()

        descs = []
        for j in range(CPW):
            descs.append(pltpu.async_copy(
                rows_v, agg_sh.at[idxs.at[j]], asem, add=True))
        for d in descs:
            d.wait()
        plsc.subcore_barrier()
        pltpu.sync_copy(agg_sh.at[pl.ds(s_ * ROWS_PER_TILE, ROWS_PER_TILE)],
                        out_hbm.at[c, pl.ds(s_ * ROWS_PER_TILE, ROWS_PER_TILE)])

    return k(dst2d, zeros, ones_rows)


_ET = 2048


def _msgs_body(attr_ref, hsrc_ref, w1t_ref, b1_ref, w2t_ref, b2_ref,
               exp_ref, red_ref, out_ref):
    eh = jnp.maximum(
        jnp.dot(attr_ref[...], w1t_ref[...],
                preferred_element_type=jnp.float32) + b1_ref[...], 0.0)
    we = jnp.dot(eh.astype(jnp.bfloat16), w2t_ref[...],
                 preferred_element_type=jnp.float32)
    hsrc = hsrc_ref[...]
    hexp = jnp.dot(hsrc.astype(jnp.bfloat16), exp_ref[...],
                   preferred_element_type=jnp.float32)
    p = hexp * we
    q = (p[:, 0:128] + p[:, 128:256] + p[:, 256:384] + p[:, 384:512]
         + p[:, 512:640] + p[:, 640:768] + p[:, 768:896] + p[:, 896:1024])
    out_ref[...] = (
        jnp.dot(q.astype(jnp.bfloat16), red_ref[...],
                preferred_element_type=jnp.float32)
        + jnp.dot(hsrc, b2_ref[...], preferred_element_type=jnp.float32))


def _tc_msgs(attr, hsrc, w1t, b1r, w2t, b2r, expc, redc):
    return pl.pallas_call(
        _msgs_body,
        grid=(EP // _ET,),
        in_specs=[
            pl.BlockSpec((_ET, EDGE_FEAT), lambda i: (i, 0)),
            pl.BlockSpec((_ET, H), lambda i: (i, 0)),
            pl.BlockSpec((EDGE_FEAT, EDGE_HID), lambda i: (0, 0)),
            pl.BlockSpec((1, EDGE_HID), lambda i: (0, 0)),
            pl.BlockSpec((EDGE_HID, H * H), lambda i: (0, 0)),
            pl.BlockSpec((H, H), lambda i: (0, 0)),
            pl.BlockSpec((H, H * H), lambda i: (0, 0)),
            pl.BlockSpec((4 * H, H), lambda i: (0, 0)),
        ],
        out_specs=pl.BlockSpec((_ET, H), lambda i: (i, 0)),
        out_shape=jax.ShapeDtypeStruct((EP, H), jnp.float32),
    )(attr, hsrc, w1t, b1r, w2t, b2r, expc, redc)


_NT = 512


def _gru_body(h_ref, a0_ref, a1_ref, d0_ref, d1_ref, rwt_ref, cb_ref,
              wir_ref, wiz_ref, win_ref, whr_ref, whz_ref, whn_ref,
              br_ref, bz_ref, bin_ref, bhn_ref, out_ref):
    h = h_ref[...]
    deg = jnp.maximum(d0_ref[...] + d1_ref[...], 1.0)
    agg = (a0_ref[...] + a1_ref[...]) / deg
    m = jnp.dot(h, rwt_ref[...], preferred_element_type=jnp.float32) \
        + agg + cb_ref[...]
    r = jax.nn.sigmoid(
        jnp.dot(m, wir_ref[...], preferred_element_type=jnp.float32)
        + jnp.dot(h, whr_ref[...], preferred_element_type=jnp.float32)
        + br_ref[...])
    z = jax.nn.sigmoid(
        jnp.dot(m, wiz_ref[...], preferred_element_type=jnp.float32)
        + jnp.dot(h, whz_ref[...], preferred_element_type=jnp.float32)
        + bz_ref[...])
    nn = jnp.tanh(
        jnp.dot(m, win_ref[...], preferred_element_type=jnp.float32)
        + bin_ref[...]
        + r * (jnp.dot(h, whn_ref[...], preferred_element_type=jnp.float32)
               + bhn_ref[...]))
    out_ref[...] = (1.0 - z) * nn + z * h


def _tc_gru(h, a0, a1, d0, d1, rwt, cb, wir, wiz, win, whr, whz, whn,
            br, bz, bin_, bhn):
    nmat = pl.BlockSpec((_NT, H), lambda i: (i, 0))
    smat = pl.BlockSpec((H, H), lambda i: (0, 0))
    svec = pl.BlockSpec((1, H), lambda i: (0, 0))
    return pl.pallas_call(
        _gru_body,
        grid=(pl.cdiv(N, _NT),),
        in_specs=[nmat, nmat, nmat, nmat, nmat, smat, svec,
                  smat, smat, smat, smat, smat, smat,
                  svec, svec, svec, svec],
        out_specs=nmat,
        out_shape=jax.ShapeDtypeStruct((N, H), jnp.float32),
    )(h, a0, a1, d0, d1, rwt, cb, wir, wiz, win, whr, whz, whn,
      br, bz, bin_, bhn)


def _s2s_body(nr_ref, bt_ref,
              wii_ref, wif_ref, wig_ref, wio_ref,
              whi_ref, whf_ref, whg_ref, who_ref,
              bi_ref, bf_ref, bg_ref, bo_ref,
              rw1_ref, rb1_ref, rw2_ref, rb2_ref, out_ref):
    nr = nr_ref[...]
    bt = bt_ref[...]
    gids = lax.broadcasted_iota(jnp.int32, (NG, 1), 0)
    maskb = bt == gids
    maskf = maskb.astype(jnp.float32)

    q_star = jnp.zeros((NG, 2 * H), jnp.float32)
    hh = jnp.zeros((NG, H), jnp.float32)
    cc = jnp.zeros((NG, H), jnp.float32)
    for _ in range(STEPS):
        def gate(wi, wh, b):
            return (jnp.dot(q_star, wi, preferred_element_type=jnp.float32)
                    + jnp.dot(hh, wh, preferred_element_type=jnp.float32)
                    + b)
        ig = jax.nn.sigmoid(gate(wii_ref[...], whi_ref[...], bi_ref[...]))
        fg = jax.nn.sigmoid(gate(wif_ref[...], whf_ref[...], bf_ref[...]))
        gg = jnp.tanh(gate(wig_ref[...], whg_ref[...], bg_ref[...]))
        og = jax.nn.sigmoid(gate(wio_ref[...], who_ref[...], bo_ref[...]))
        cc = fg * cc + ig * gg
        hh = og * jnp.tanh(cc)
        q = hh
        d = lax.dot_general(q, nr, (((1,), (1,)), ((), ())),
                            preferred_element_type=jnp.float32)
        neg = jnp.float32(-1e30)
        emax = jnp.max(jnp.where(maskb, d, neg), axis=1, keepdims=True)
        emax = jnp.where(emax > jnp.float32(-1e29), emax, 0.0)
        ex = jnp.where(maskb, jnp.exp(d - emax), 0.0)
        denom = jnp.sum(ex, axis=1, keepdims=True)
        a = ex / jnp.maximum(denom, jnp.float32(1e-30))
        r_read = jnp.dot(a, nr, preferred_element_type=jnp.float32)
        q_star = jnp.concatenate([q, r_read], axis=1)
    hid = jnp.maximum(
        jnp.dot(q_star, rw1_ref[...], preferred_element_type=jnp.float32)
        + rb1_ref[...], 0.0)
    out_ref[...] = jnp.dot(hid, rw2_ref[...],
                           preferred_element_type=jnp.float32) + rb2_ref[...]


def _tc_set2set(nr, bt2d, lstm_wih, lstm_whh, lstm_bih, lstm_bhh,
                ro_w1, ro_b1, ro_w2, ro_b2):
    wis = [lstm_wih[k * H:(k + 1) * H].T for k in range(4)]
    whs = [lstm_whh[k * H:(k + 1) * H].T for k in range(4)]
    bs = [(lstm_bih[k * H:(k + 1) * H]
           + lstm_bhh[k * H:(k + 1) * H]).reshape(1, H) for k in range(4)]
    return pl.pallas_call(
        _s2s_body,
        out_shape=jax.ShapeDtypeStruct((NG, H), jnp.float32),
    )(nr, bt2d, *wis, *whs, *bs,
      ro_w1.T, ro_b1.reshape(1, H), ro_w2.T, ro_b2.reshape(1, H))


def kernel(x, edge_index, edge_attr, batch, W1, b1, W2, b2, root_w, conv_b,
           gru_wih, gru_whh, gru_bih, gru_bhh, lstm_wih, lstm_whh, lstm_bih,
           lstm_bhh, ro_w1, ro_b1, ro_w2, ro_b2):
    npad = EP - E
    src2d = jnp.pad(edge_index[0], (0, npad)).reshape(NCHUNK, EC)
    dst2d = jnp.pad(edge_index[1], (0, npad),
                    constant_values=TRASH).reshape(NCHUNK, EC)
    attr_pad = jnp.pad(edge_attr, ((0, npad), (0, 0)))
    zeros_np = jnp.zeros((NP_, H), jnp.float32)
    ones_rows = jnp.ones((EC, H), jnp.float32)

    h = jnp.pad(x, ((0, 0), (0, H - FEAT)))

    w1t = W1.T
    b1r = b1.reshape(1, EDGE_HID)
    w2t = W2.T.astype(jnp.bfloat16)
    b2r = b2.reshape(H, H)
    expc = jnp.asarray(_EXP, dtype=jnp.bfloat16)
    redc = jnp.asarray(_RED, dtype=jnp.bfloat16)

    rwt = root_w.T
    cb = conv_b.reshape(1, H)
    wir, wiz, win = (gru_wih[k * H:(k + 1) * H].T for k in range(3))
    whr, whz, whn = (gru_whh[k * H:(k + 1) * H].T for k in range(3))
    br = (gru_bih[:H] + gru_bhh[:H]).reshape(1, H)
    bz = (gru_bih[H:2 * H] + gru_bhh[H:2 * H]).reshape(1, H)
    bin_ = gru_bih[2 * H:].reshape(1, H)
    bhn = gru_bhh[2 * H:].reshape(1, H)

    d0 = d1 = None
    for layer in range(LAYERS):
        if layer == 0:
            hsrc, degp = _sc_gather_deg(h, src2d, dst2d, zeros_np, ones_rows)
            d0, d1 = degp[0, :N], degp[1, :N]
        else:
            hsrc = _sc_gather(h, src2d)
        msgs = _tc_msgs(attr_pad, hsrc, w1t, b1r, w2t, b2r, expc, redc)
        aggp = _sc_scatter(msgs, dst2d, zeros_np)
        h = _tc_gru(h, aggp[0, :N], aggp[1, :N], d0, d1, rwt, cb,
                    wir, wiz, win, whr, whz, whn, br, bz, bin_, bhn)

    node_repr = h
    graph_repr = _tc_set2set(node_repr, batch.reshape(1, N),
                             lstm_wih, lstm_whh, lstm_bih, lstm_bhh,
                             ro_w1, ro_b1, ro_w2, ro_b2)
    return node_repr, graph_repr


# speedup vs baseline: 1.1095x; 1.1095x over previous
import functools

import numpy as np
import jax
import jax.numpy as jnp
from jax import lax
from jax.experimental import pallas as pl
from jax.experimental.pallas import tpu as pltpu
from jax.experimental.pallas import tpu_sc as plsc

N = 10000
E = 160000
NG = 64
H = 32
FEAT = 16
EDGE_FEAT = 4
EDGE_HID = 128
LAYERS = 3
STEPS = 3

NP_ = 10240
EC = 128
NW = 32
CPW = 40
NCHUNK = NW * CPW
EP = NCHUNK * EC
SUP = 8
NSUP = CPW // SUP
TRASH = NP_ - 1
ROWS_PER_TILE = NP_ // 16

_EXP = np.kron(np.eye(H, dtype=np.float32), np.ones((1, H), dtype=np.float32))
_RED = np.tile(np.eye(H, dtype=np.float32), (4, 1))


def _sc_mesh():
    return plsc.VectorSubcoreMesh(core_axis_name="c", subcore_axis_name="s",
                                  num_cores=2, num_subcores=16)


_SC_PARAMS = pltpu.CompilerParams(use_tc_tiling_on_sc=False)


def _sc_gather(h, src2d):
    @functools.partial(
        pl.kernel,
        out_type=jax.ShapeDtypeStruct((EP, H), jnp.float32),
        mesh=_sc_mesh(),
        scratch_types=[
            pltpu.VMEM((CPW, EC), jnp.int32),
            pltpu.VMEM((2, SUP * EC, H), jnp.float32),
            pltpu.VMEM_SHARED((N, H), jnp.float32),
            pltpu.SemaphoreType.DMA,
            pltpu.SemaphoreType.DMA,
        ],
        compiler_params=_SC_PARAMS,
    )
    def k(h_hbm, src_hbm, out_hbm, idxs, rows, h_sh, gsem, ssem):
        wid = lax.axis_index("s") * 2 + lax.axis_index("c")
        base = wid * CPW

        @pl.when(lax.axis_index("s") == 0)
        def _():
            pltpu.sync_copy(h_hbm, h_sh)

        pltpu.sync_copy(src_hbm.at[pl.ds(base, CPW)], idxs)
        plsc.subcore_barrier()
        stores = [None, None]
        for sc in range(NSUP):
            b = sc % 2
            if stores[b] is not None:
                stores[b].wait()
            descs = []
            for jj in range(SUP):
                descs.append(pltpu.async_copy(
                    h_sh.at[idxs.at[sc * SUP + jj]],
                    rows.at[b, pl.ds(jj * EC, EC)], gsem))
            for d in descs:
                d.wait()
            st = pltpu.async_copy(
                rows.at[b],
                out_hbm.at[pl.ds((base + sc * SUP) * EC, SUP * EC)], ssem)
            stores[b] = st
        for st in stores:
            if st is not None:
                st.wait()

    return k(h, src2d)


def _sc_gather_deg(h, src2d, dst2d, zeros, ones_rows):
    @functools.partial(
        pl.kernel,
        out_type=(jax.ShapeDtypeStruct((EP, H), jnp.float32),
                  jax.ShapeDtypeStruct((2, NP_, H), jnp.float32)),
        mesh=_sc_mesh(),
        scratch_types=[
            pltpu.VMEM((CPW, EC), jnp.int32),
            pltpu.VMEM((CPW, EC), jnp.int32),
            pltpu.VMEM((2, SUP * EC, H), jnp.float32),
            pltpu.VMEM((EC, H), jnp.float32),
            pltpu.VMEM_SHARED((NP_, H), jnp.float32),
            pltpu.VMEM_SHARED((N, H), jnp.float32),
            pltpu.SemaphoreType.DMA,
            pltpu.SemaphoreType.DMA,
            pltpu.SemaphoreType.DMA,
        ],
        compiler_params=_SC_PARAMS,
    )
    def k(h_hbm, src_hbm, dst_hbm, zeros_hbm, ones_hbm, out_hbm, deg_hbm,
          idxs, didxs, rows, ones_v, agg_sh, h_sh, gsem, ssem, asem):
        c = lax.axis_index("c")
        s_ = lax.axis_index("s")
        wid = s_ * 2 + c
        base = wid * CPW
        dbase = (c * 16 + s_) * CPW

        @pl.when(s_ == 0)
        def _():
            pltpu.sync_copy(zeros_hbm, agg_sh)

        @pl.when(s_ == 1)
        def _():
            pltpu.sync_copy(h_hbm, h_sh)

        pltpu.sync_copy(src_hbm.at[pl.ds(base, CPW)], idxs)
        pltpu.sync_copy(dst_hbm.at[pl.ds(dbase, CPW)], didxs)
        pltpu.sync_copy(ones_hbm, ones_v)
        plsc.subcore_barrier()

        adds = []
        for j in range(CPW):
            adds.append(pltpu.async_copy(
                ones_v, agg_sh.at[didxs.at[j]], asem, add=True))

        stores = [None, None]
        for sc in range(NSUP):
            b = sc % 2
            if stores[b] is not None:
                stores[b].wait()
            descs = []
            for jj in range(SUP):
                descs.append(pltpu.async_copy(
                    h_sh.at[idxs.at[sc * SUP + jj]],
                    rows.at[b, pl.ds(jj * EC, EC)], gsem))
            for d in descs:
                d.wait()
            st = pltpu.async_copy(
                rows.at[b],
                out_hbm.at[pl.ds((base + sc * SUP) * EC, SUP * EC)], ssem)
            stores[b] = st
        for st in stores:
            if st is not None:
                st.wait()
        for d in adds:
            d.wait()
        plsc.subcore_barrier()
        pltpu.sync_copy(agg_sh.at[pl.ds(s_ * ROWS_PER_TILE, ROWS_PER_TILE)],
                        deg_hbm.at[c, pl.ds(s_ * ROWS_PER_TILE, ROWS_PER_TILE)])

    return k(h, src2d, dst2d, zeros, ones_rows)


def _sc_scatter(msgs, dst2d, zeros):
    @functools.partial(
        pl.kernel,
        out_type=jax.ShapeDtypeStruct((2, NP_, H), jnp.float32),
        mesh=_sc_mesh(),
        scratch_types=[
            pltpu.VMEM((CPW, EC), jnp.int32),
            pltpu.VMEM((2, SUP * EC, H), jnp.float32),
            pltpu.VMEM_SHARED((NP_, H), jnp.float32),
            pltpu.SemaphoreType.DMA,
            pltpu.SemaphoreType.DMA,
        ],
        compiler_params=_SC_PARAMS,
    )
    def k(msgs_hbm, dst_hbm, zeros_hbm, out_hbm, idxs, rows, agg_sh,
          lsem, asem):
        c = lax.axis_index("c")
        s_ = lax.axis_index("s")
        base = (c * 16 + s_) * CPW

        @pl.when(s_ == 0)
        def _():
            pltpu.sync_copy(zeros_hbm, agg_sh)

        pltpu.sync_copy(dst_hbm.at[pl.ds(base, CPW)], idxs)
        plsc.subcore_barrier()

        loads = [None, None]
        adds = [None] * NSUP
        loads[0] = pltpu.async_copy(
            msgs_hbm.at[pl.ds(base * EC, SUP * EC)], rows.at[0], lsem)
        for sc in range(NSUP):
            b = sc % 2
            loads[b].wait()
            if sc + 1 < NSUP:
                if adds[sc - 1] is not None:
                    for d in adds[sc - 1]:
                        d.wait()
                    adds[sc - 1] = None
                loads[(sc + 1) % 2] = pltpu.async_copy(
                    msgs_hbm.at[pl.ds((base + (sc + 1) * SUP) * EC, SUP * EC)],
                    rows.at[(sc + 1) % 2], lsem)
            batch = []
            for jj in range(SUP):
                batch.append(pltpu.async_copy(
                    rows.at[b, pl.ds(jj * EC, EC)],
                    agg_sh.at[idxs.at[sc * SUP + jj]], asem, add=True))
            adds[sc] = batch
        for batch in adds:
            if batch is not None:
                for d in batch:
                    d.wait()
        plsc.subcore_barrier()
        pltpu.sync_copy(agg_sh.at[pl.ds(s_ * ROWS_PER_TILE, ROWS_PER_TILE)],
                        out_hbm.at[c, pl.ds(s_ * ROWS_PER_TILE, ROWS_PER_TILE)])

    return k(msgs, dst2d, zeros)


def _sc_degree(dst2d, zeros, ones_rows):
    @functools.partial(
        pl.kernel,
        out_type=jax.ShapeDtypeStruct((2, NP_, H), jnp.float32),
        mesh=_sc_mesh(),
        scratch_types=[
            pltpu.VMEM((CPW, EC), jnp.int32),
            pltpu.VMEM((EC, H), jnp.float32),
            pltpu.VMEM_SHARED((NP_, H), jnp.float32),
            pltpu.SemaphoreType.DMA,
        ],
        compiler_params=_SC_PARAMS,
    )
    def k(dst_hbm, zeros_hbm, ones_hbm, out_hbm, idxs, rows_v, agg_sh, asem):
        c = lax.axis_index("c")
        s_ = lax.axis_index("s")
        base = (c * 16 + s_) * CPW

        @pl.when(s_ == 0)
        def _():
            pltpu.sync_copy(zeros_hbm, agg_sh)

        pltpu.sync_copy(dst_hbm.at[pl.ds(base, CPW)], idxs)
        pltpu.sync_copy(ones_hbm, rows_v)
        plsc.subcore_barrier()

        descs = []
        for j in range(CPW):
            descs.append(pltpu.async_copy(
                rows_v, agg_sh.at[idxs.at[j]], asem, add=True))
        for d in descs:
            d.wait()
        plsc.subcore_barrier()
        pltpu.sync_copy(agg_sh.at[pl.ds(s_ * ROWS_PER_TILE, ROWS_PER_TILE)],
                        out_hbm.at[c, pl.ds(s_ * ROWS_PER_TILE, ROWS_PER_TILE)])

    return k(dst2d, zeros, ones_rows)


_ET = 2048


def _msgs_body(attr_ref, hsrc_ref, w1t_ref, b1_ref, w2t_ref, b2_ref,
               exp_ref, red_ref, out_ref):
    eh = jnp.maximum(
        jnp.dot(attr_ref[...], w1t_ref[...],
                preferred_element_type=jnp.float32) + b1_ref[...], 0.0)
    we = jnp.dot(eh.astype(jnp.bfloat16), w2t_ref[...],
                 preferred_element_type=jnp.float32)
    hsrc = hsrc_ref[...]
    hexp = jnp.dot(hsrc.astype(jnp.bfloat16), exp_ref[...],
                   preferred_element_type=jnp.float32)
    p = hexp * we
    q = (p[:, 0:128] + p[:, 128:256] + p[:, 256:384] + p[:, 384:512]
         + p[:, 512:640] + p[:, 640:768] + p[:, 768:896] + p[:, 896:1024])
    out_ref[...] = (
        jnp.dot(q.astype(jnp.bfloat16), red_ref[...],
                preferred_element_type=jnp.float32)
        + jnp.dot(hsrc, b2_ref[...], preferred_element_type=jnp.float32))


def _tc_msgs(attr, hsrc, w1t, b1r, w2t, b2r, expc, redc):
    return pl.pallas_call(
        _msgs_body,
        grid=(EP // _ET,),
        in_specs=[
            pl.BlockSpec((_ET, EDGE_FEAT), lambda i: (i, 0)),
            pl.BlockSpec((_ET, H), lambda i: (i, 0)),
            pl.BlockSpec((EDGE_FEAT, EDGE_HID), lambda i: (0, 0)),
            pl.BlockSpec((1, EDGE_HID), lambda i: (0, 0)),
            pl.BlockSpec((EDGE_HID, H * H), lambda i: (0, 0)),
            pl.BlockSpec((H, H), lambda i: (0, 0)),
            pl.BlockSpec((H, H * H), lambda i: (0, 0)),
            pl.BlockSpec((4 * H, H), lambda i: (0, 0)),
        ],
        out_specs=pl.BlockSpec((_ET, H), lambda i: (i, 0)),
        out_shape=jax.ShapeDtypeStruct((EP, H), jnp.float32),
    )(attr, hsrc, w1t, b1r, w2t, b2r, expc, redc)


_NT = 512


def _gru_body(h_ref, a0_ref, a1_ref, d0_ref, d1_ref, rwt_ref, cb_ref,
              wir_ref, wiz_ref, win_ref, whr_ref, whz_ref, whn_ref,
              br_ref, bz_ref, bin_ref, bhn_ref, out_ref):
    h = h_ref[...]
    deg = jnp.maximum(d0_ref[...] + d1_ref[...], 1.0)
    agg = (a0_ref[...] + a1_ref[...]) / deg
    m = jnp.dot(h, rwt_ref[...], preferred_element_type=jnp.float32) \
        + agg + cb_ref[...]
    r = jax.nn.sigmoid(
        jnp.dot(m, wir_ref[...], preferred_element_type=jnp.float32)
        + jnp.dot(h, whr_ref[...], preferred_element_type=jnp.float32)
        + br_ref[...])
    z = jax.nn.sigmoid(
        jnp.dot(m, wiz_ref[...], preferred_element_type=jnp.float32)
        + jnp.dot(h, whz_ref[...], preferred_element_type=jnp.float32)
        + bz_ref[...])
    nn = jnp.tanh(
        jnp.dot(m, win_ref[...], preferred_element_type=jnp.float32)
        + bin_ref[...]
        + r * (jnp.dot(h, whn_ref[...], preferred_element_type=jnp.float32)
               + bhn_ref[...]))
    out_ref[...] = (1.0 - z) * nn + z * h


def _tc_gru(h, a0, a1, d0, d1, rwt, cb, wir, wiz, win, whr, whz, whn,
            br, bz, bin_, bhn):
    nmat = pl.BlockSpec((_NT, H), lambda i: (i, 0))
    smat = pl.BlockSpec((H, H), lambda i: (0, 0))
    svec = pl.BlockSpec((1, H), lambda i: (0, 0))
    return pl.pallas_call(
        _gru_body,
        grid=(pl.cdiv(N, _NT),),
        in_specs=[nmat, nmat, nmat, nmat, nmat, smat, svec,
                  smat, smat, smat, smat, smat, smat,
                  svec, svec, svec, svec],
        out_specs=nmat,
        out_shape=jax.ShapeDtypeStruct((N, H), jnp.float32),
    )(h, a0, a1, d0, d1, rwt, cb, wir, wiz, win, whr, whz, whn,
      br, bz, bin_, bhn)


def _s2s_body(nr_ref, bt_ref,
              wii_ref, wif_ref, wig_ref, wio_ref,
              whi_ref, whf_ref, whg_ref, who_ref,
              bi_ref, bf_ref, bg_ref, bo_ref,
              rw1_ref, rb1_ref, rw2_ref, rb2_ref, out_ref):
    nr = nr_ref[...]
    bt = bt_ref[...]
    gids = lax.broadcasted_iota(jnp.int32, (NG, 1), 0)
    maskb = bt == gids
    maskf = maskb.astype(jnp.float32)

    q_star = jnp.zeros((NG, 2 * H), jnp.float32)
    hh = jnp.zeros((NG, H), jnp.float32)
    cc = jnp.zeros((NG, H), jnp.float32)
    for _ in range(STEPS):
        def gate(wi, wh, b):
            return (jnp.dot(q_star, wi, preferred_element_type=jnp.float32)
                    + jnp.dot(hh, wh, preferred_element_type=jnp.float32)
                    + b)
        ig = jax.nn.sigmoid(gate(wii_ref[...], whi_ref[...], bi_ref[...]))
        fg = jax.nn.sigmoid(gate(wif_ref[...], whf_ref[...], bf_ref[...]))
        gg = jnp.tanh(gate(wig_ref[...], whg_ref[...], bg_ref[...]))
        og = jax.nn.sigmoid(gate(wio_ref[...], who_ref[...], bo_ref[...]))
        cc = fg * cc + ig * gg
        hh = og * jnp.tanh(cc)
        q = hh
        d = lax.dot_general(q, nr, (((1,), (1,)), ((), ())),
                            preferred_element_type=jnp.float32)
        neg = jnp.float32(-1e30)
        emax = jnp.max(jnp.where(maskb, d, neg), axis=1, keepdims=True)
        emax = jnp.where(emax > jnp.float32(-1e29), emax, 0.0)
        ex = jnp.where(maskb, jnp.exp(d - emax), 0.0)
        denom = jnp.sum(ex, axis=1, keepdims=True)
        a = ex / jnp.maximum(denom, jnp.float32(1e-30))
        r_read = jnp.dot(a, nr, preferred_element_type=jnp.float32)
        q_star = jnp.concatenate([q, r_read], axis=1)
    hid = jnp.maximum(
        jnp.dot(q_star, rw1_ref[...], preferred_element_type=jnp.float32)
        + rb1_ref[...], 0.0)
    out_ref[...] = jnp.dot(hid, rw2_ref[...],
                           preferred_element_type=jnp.float32) + rb2_ref[...]


def _tc_set2set(nr, bt2d, lstm_wih, lstm_whh, lstm_bih, lstm_bhh,
                ro_w1, ro_b1, ro_w2, ro_b2):
    wis = [lstm_wih[k * H:(k + 1) * H].T for k in range(4)]
    whs = [lstm_whh[k * H:(k + 1) * H].T for k in range(4)]
    bs = [(lstm_bih[k * H:(k + 1) * H]
           + lstm_bhh[k * H:(k + 1) * H]).reshape(1, H) for k in range(4)]
    return pl.pallas_call(
        _s2s_body,
        out_shape=jax.ShapeDtypeStruct((NG, H), jnp.float32),
    )(nr, bt2d, *wis, *whs, *bs,
      ro_w1.T, ro_b1.reshape(1, H), ro_w2.T, ro_b2.reshape(1, H))


def kernel(x, edge_index, edge_attr, batch, W1, b1, W2, b2, root_w, conv_b,
           gru_wih, gru_whh, gru_bih, gru_bhh, lstm_wih, lstm_whh, lstm_bih,
           lstm_bhh, ro_w1, ro_b1, ro_w2, ro_b2):
    npad = EP - E
    src2d = jnp.pad(edge_index[0], (0, npad)).reshape(NCHUNK, EC)
    dst2d = jnp.pad(edge_index[1], (0, npad),
                    constant_values=TRASH).reshape(NCHUNK, EC)
    attr_pad = jnp.pad(edge_attr, ((0, npad), (0, 0)))
    zeros_np = jnp.zeros((NP_, H), jnp.float32)
    ones_rows = jnp.ones((EC, H), jnp.float32)

    h = jnp.pad(x, ((0, 0), (0, H - FEAT)))

    w1t = W1.T
    b1r = b1.reshape(1, EDGE_HID)
    w2t = W2.T.astype(jnp.bfloat16)
    b2r = b2.reshape(H, H)
    expc = jnp.asarray(_EXP, dtype=jnp.bfloat16)
    redc = jnp.asarray(_RED, dtype=jnp.bfloat16)

    rwt = root_w.T
    cb = conv_b.reshape(1, H)
    wir, wiz, win = (gru_wih[k * H:(k + 1) * H].T for k in range(3))
    whr, whz, whn = (gru_whh[k * H:(k + 1) * H].T for k in range(3))
    br = (gru_bih[:H] + gru_bhh[:H]).reshape(1, H)
    bz = (gru_bih[H:2 * H] + gru_bhh[H:2 * H]).reshape(1, H)
    bin_ = gru_bih[2 * H:].reshape(1, H)
    bhn = gru_bhh[2 * H:].reshape(1, H)

    d0 = d1 = None
    for layer in range(LAYERS):
        if layer == 0:
            hsrc, degp = _sc_gather_deg(h, src2d, dst2d, zeros_np, ones_rows)
            d0, d1 = degp[0, :N], degp[1, :N]
        else:
            hsrc = _sc_gather(h, src2d)
        msgs = _tc_msgs(attr_pad, hsrc, w1t, b1r, w2t, b2r, expc, redc)
        aggp = _sc_scatter(msgs, dst2d, zeros_np)
        h = _tc_gru(h, aggp[0, :N], aggp[1, :N], d0, d1, rwt, cb,
                    wir, wiz, win, whr, whz, whn, br, bz, bin_, bhn)

    node_repr = h
    graph_repr = _tc_set2set(node_repr, batch.reshape(1, N),
                             lstm_wih, lstm_whh, lstm_bih, lstm_bhh,
                             ro_w1, ro_b1, ro_w2, ro_b2)
    return node_repr, graph_repr


# speedup vs baseline: 1.1134x; 1.0035x over previous
import functools

import numpy as np
import jax
import jax.numpy as jnp
from jax import lax
from jax.experimental import pallas as pl
from jax.experimental.pallas import tpu as pltpu
from jax.experimental.pallas import tpu_sc as plsc

N = 10000
E = 160000
NG = 64
H = 32
FEAT = 16
EDGE_FEAT = 4
EDGE_HID = 128
LAYERS = 3
STEPS = 3

NP_ = 10240
EC = 128
NW = 32
CPW = 40
NCHUNK = NW * CPW
EP = NCHUNK * EC
SUP = 8
NSUP = CPW // SUP
TRASH = NP_ - 1
ROWS_PER_TILE = NP_ // 16

_EXP = np.kron(np.eye(H, dtype=np.float32), np.ones((1, H), dtype=np.float32))
_RED = np.tile(np.eye(H, dtype=np.float32), (4, 1))


def _sc_mesh():
    return plsc.VectorSubcoreMesh(core_axis_name="c", subcore_axis_name="s",
                                  num_cores=2, num_subcores=16)


_SC_PARAMS = pltpu.CompilerParams(use_tc_tiling_on_sc=False)


def _sc_gather(h, src2d):
    @functools.partial(
        pl.kernel,
        out_type=jax.ShapeDtypeStruct((EP, H), jnp.float32),
        mesh=_sc_mesh(),
        scratch_types=[
            pltpu.VMEM((CPW, EC), jnp.int32),
            pltpu.VMEM((2, SUP * EC, H), jnp.float32),
            pltpu.VMEM_SHARED((N, H), jnp.float32),
            pltpu.SemaphoreType.DMA,
            pltpu.SemaphoreType.DMA,
        ],
        compiler_params=_SC_PARAMS,
    )
    def k(h_hbm, src_hbm, out_hbm, idxs, rows, h_sh, gsem, ssem):
        wid = lax.axis_index("s") * 2 + lax.axis_index("c")
        base = wid * CPW

        @pl.when(lax.axis_index("s") == 0)
        def _():
            pltpu.sync_copy(h_hbm, h_sh)

        pltpu.sync_copy(src_hbm.at[pl.ds(base, CPW)], idxs)
        plsc.subcore_barrier()
        stores = [None, None]
        for sc in range(NSUP):
            b = sc % 2
            if stores[b] is not None:
                stores[b].wait()
            descs = []
            for jj in range(SUP):
                descs.append(pltpu.async_copy(
                    h_sh.at[idxs.at[sc * SUP + jj]],
                    rows.at[b, pl.ds(jj * EC, EC)], gsem))
            for d in descs:
                d.wait()
            st = pltpu.async_copy(
                rows.at[b],
                out_hbm.at[pl.ds((base + sc * SUP) * EC, SUP * EC)], ssem)
            stores[b] = st
        for st in stores:
            if st is not None:
                st.wait()

    return k(h, src2d)


def _sc_gather_deg(h, src2d, dst2d, zeros, ones_rows):
    @functools.partial(
        pl.kernel,
        out_type=(jax.ShapeDtypeStruct((EP, H), jnp.float32),
                  jax.ShapeDtypeStruct((2, NP_, H), jnp.float32)),
        mesh=_sc_mesh(),
        scratch_types=[
            pltpu.VMEM((CPW, EC), jnp.int32),
            pltpu.VMEM((CPW, EC), jnp.int32),
            pltpu.VMEM((2, SUP * EC, H), jnp.float32),
            pltpu.VMEM((EC, H), jnp.float32),
            pltpu.VMEM_SHARED((NP_, H), jnp.float32),
            pltpu.VMEM_SHARED((N, H), jnp.float32),
            pltpu.SemaphoreType.DMA,
            pltpu.SemaphoreType.DMA,
            pltpu.SemaphoreType.DMA,
        ],
        compiler_params=_SC_PARAMS,
    )
    def k(h_hbm, src_hbm, dst_hbm, zeros_hbm, ones_hbm, out_hbm, deg_hbm,
          idxs, didxs, rows, ones_v, agg_sh, h_sh, gsem, ssem, asem):
        c = lax.axis_index("c")
        s_ = lax.axis_index("s")
        wid = s_ * 2 + c
        base = wid * CPW
        dbase = (c * 16 + s_) * CPW

        @pl.when(s_ == 0)
        def _():
            pltpu.sync_copy(zeros_hbm, agg_sh)

        @pl.when(s_ == 1)
        def _():
            pltpu.sync_copy(h_hbm, h_sh)

        pltpu.sync_copy(src_hbm.at[pl.ds(base, CPW)], idxs)
        pltpu.sync_copy(dst_hbm.at[pl.ds(dbase, CPW)], didxs)
        pltpu.sync_copy(ones_hbm, ones_v)
        plsc.subcore_barrier()

        adds = []
        for j in range(CPW):
            adds.append(pltpu.async_copy(
                ones_v, agg_sh.at[didxs.at[j]], asem, add=True))

        stores = [None, None]
        for sc in range(NSUP):
            b = sc % 2
            if stores[b] is not None:
                stores[b].wait()
            descs = []
            for jj in range(SUP):
                descs.append(pltpu.async_copy(
                    h_sh.at[idxs.at[sc * SUP + jj]],
                    rows.at[b, pl.ds(jj * EC, EC)], gsem))
            for d in descs:
                d.wait()
            st = pltpu.async_copy(
                rows.at[b],
                out_hbm.at[pl.ds((base + sc * SUP) * EC, SUP * EC)], ssem)
            stores[b] = st
        for st in stores:
            if st is not None:
                st.wait()
        for d in adds:
            d.wait()
        plsc.subcore_barrier()
        pltpu.sync_copy(agg_sh.at[pl.ds(s_ * ROWS_PER_TILE, ROWS_PER_TILE)],
                        deg_hbm.at[c, pl.ds(s_ * ROWS_PER_TILE, ROWS_PER_TILE)])

    return k(h, src2d, dst2d, zeros, ones_rows)


def _sc_scatter(msgs, dst2d, zeros):
    @functools.partial(
        pl.kernel,
        out_type=jax.ShapeDtypeStruct((2, NP_, H), jnp.float32),
        mesh=_sc_mesh(),
        scratch_types=[
            pltpu.VMEM((CPW, EC), jnp.int32),
            pltpu.VMEM((2, SUP * EC, H), jnp.float32),
            pltpu.VMEM_SHARED((NP_, H), jnp.float32),
            pltpu.SemaphoreType.DMA,
            pltpu.SemaphoreType.DMA,
        ],
        compiler_params=_SC_PARAMS,
    )
    def k(msgs_hbm, dst_hbm, zeros_hbm, out_hbm, idxs, rows, agg_sh,
          lsem, asem):
        c = lax.axis_index("c")
        s_ = lax.axis_index("s")
        base = (c * 16 + s_) * CPW

        @pl.when(s_ == 0)
        def _():
            pltpu.sync_copy(zeros_hbm, agg_sh)

        pltpu.sync_copy(dst_hbm.at[pl.ds(base, CPW)], idxs)
        plsc.subcore_barrier()

        loads = [None, None]
        adds = [None] * NSUP
        loads[0] = pltpu.async_copy(
            msgs_hbm.at[pl.ds(base * EC, SUP * EC)], rows.at[0], lsem)
        for sc in range(NSUP):
            b = sc % 2
            loads[b].wait()
            if sc + 1 < NSUP:
                if adds[sc - 1] is not None:
                    for d in adds[sc - 1]:
                        d.wait()
                    adds[sc - 1] = None
                loads[(sc + 1) % 2] = pltpu.async_copy(
                    msgs_hbm.at[pl.ds((base + (sc + 1) * SUP) * EC, SUP * EC)],
                    rows.at[(sc + 1) % 2], lsem)
            batch = []
            for jj in range(SUP):
                batch.append(pltpu.async_copy(
                    rows.at[b, pl.ds(jj * EC, EC)],
                    agg_sh.at[idxs.at[sc * SUP + jj]], asem, add=True))
            adds[sc] = batch
        for batch in adds:
            if batch is not None:
                for d in batch:
                    d.wait()
        plsc.subcore_barrier()
        pltpu.sync_copy(agg_sh.at[pl.ds(s_ * ROWS_PER_TILE, ROWS_PER_TILE)],
                        out_hbm.at[c, pl.ds(s_ * ROWS_PER_TILE, ROWS_PER_TILE)])

    return k(msgs, dst2d, zeros)


def _sc_degree(dst2d, zeros, ones_rows):
    @functools.partial(
        pl.kernel,
        out_type=jax.ShapeDtypeStruct((2, NP_, H), jnp.float32),
        mesh=_sc_mesh(),
        scratch_types=[
            pltpu.VMEM((CPW, EC), jnp.int32),
            pltpu.VMEM((EC, H), jnp.float32),
            pltpu.VMEM_SHARED((NP_, H), jnp.float32),
            pltpu.SemaphoreType.DMA,
        ],
        compiler_params=_SC_PARAMS,
    )
    def k(dst_hbm, zeros_hbm, ones_hbm, out_hbm, idxs, rows_v, agg_sh, asem):
        c = lax.axis_index("c")
        s_ = lax.axis_index("s")
        base = (c * 16 + s_) * CPW

        @pl.when(s_ == 0)
        def _():
            pltpu.sync_copy(zeros_hbm, agg_sh)

        pltpu.sync_copy(dst_hbm.at[pl.ds(base, CPW)], idxs)
        pltpu.sync_copy(ones_hbm, rows_v)
        plsc.subcore_barrier()

        descs = []
        for j in range(CPW):
            descs.append(pltpu.async_copy(
                rows_v, agg_sh.at[idxs.at[j]], asem, add=True))
        for d in descs:
            d.wait()
        plsc.subcore_barrier()
        pltpu.sync_copy(agg_sh.at[pl.ds(s_ * ROWS_PER_TILE, ROWS_PER_TILE)],
                        out_hbm.at[c, pl.ds(s_ * ROWS_PER_TILE, ROWS_PER_TILE)])

    return k(dst2d, zeros, ones_rows)


_ET = 2048


def _msgs_body(attr_ref, hsrc_ref, w1t_ref, b1_ref, w2t_ref, b2_ref,
               exp_ref, red_ref, out_ref):
    eh = jnp.maximum(
        jnp.dot(attr_ref[...], w1t_ref[...],
                preferred_element_type=jnp.float32) + b1_ref[...], 0.0)
    we = jnp.dot(eh.astype(jnp.bfloat16), w2t_ref[...],
                 preferred_element_type=jnp.float32)
    hsrc = hsrc_ref[...]
    hexp = jnp.dot(hsrc.astype(jnp.bfloat16), exp_ref[...],
                   preferred_element_type=jnp.float32)
    p = hexp * we
    q = (p[:, 0:128] + p[:, 128:256] + p[:, 256:384] + p[:, 384:512]
         + p[:, 512:640] + p[:, 640:768] + p[:, 768:896] + p[:, 896:1024])
    out_ref[...] = (
        jnp.dot(q.astype(jnp.bfloat16), red_ref[...],
                preferred_element_type=jnp.float32)
        + jnp.dot(hsrc, b2_ref[...], preferred_element_type=jnp.float32))


def _tc_msgs(attr, hsrc, w1t, b1r, w2t, b2r, expc, redc):
    return pl.pallas_call(
        _msgs_body,
        grid=(EP // _ET,),
        in_specs=[
            pl.BlockSpec((_ET, EDGE_FEAT), lambda i: (i, 0)),
            pl.BlockSpec((_ET, H), lambda i: (i, 0)),
            pl.BlockSpec((EDGE_FEAT, EDGE_HID), lambda i: (0, 0)),
            pl.BlockSpec((1, EDGE_HID), lambda i: (0, 0)),
            pl.BlockSpec((EDGE_HID, H * H), lambda i: (0, 0)),
            pl.BlockSpec((H, H), lambda i: (0, 0)),
            pl.BlockSpec((H, H * H), lambda i: (0, 0)),
            pl.BlockSpec((4 * H, H), lambda i: (0, 0)),
        ],
        out_specs=pl.BlockSpec((_ET, H), lambda i: (i, 0)),
        out_shape=jax.ShapeDtypeStruct((EP, H), jnp.float32),
    )(attr, hsrc, w1t, b1r, w2t, b2r, expc, redc)


_NT = 512


def _gru_body(h_ref, a0_ref, a1_ref, d0_ref, d1_ref, rwt_ref, cb_ref,
              wir_ref, wiz_ref, win_ref, whr_ref, whz_ref, whn_ref,
              br_ref, bz_ref, bin_ref, bhn_ref, out_ref):
    h = h_ref[...]
    deg = jnp.maximum(d0_ref[...] + d1_ref[...], 1.0)
    agg = (a0_ref[...] + a1_ref[...]) / deg
    m = jnp.dot(h, rwt_ref[...], preferred_element_type=jnp.float32) \
        + agg + cb_ref[...]
    r = jax.nn.sigmoid(
        jnp.dot(m, wir_ref[...], preferred_element_type=jnp.float32)
        + jnp.dot(h, whr_ref[...], preferred_element_type=jnp.float32)
        + br_ref[...])
    z = jax.nn.sigmoid(
        jnp.dot(m, wiz_ref[...], preferred_element_type=jnp.float32)
        + jnp.dot(h, whz_ref[...], preferred_element_type=jnp.float32)
        + bz_ref[...])
    nn = jnp.tanh(
        jnp.dot(m, win_ref[...], preferred_element_type=jnp.float32)
        + bin_ref[...]
        + r * (jnp.dot(h, whn_ref[...], preferred_element_type=jnp.float32)
               + bhn_ref[...]))
    out_ref[...] = (1.0 - z) * nn + z * h


def _tc_gru(h, a0, a1, d0, d1, rwt, cb, wir, wiz, win, whr, whz, whn,
            br, bz, bin_, bhn):
    nmat = pl.BlockSpec((_NT, H), lambda i: (i, 0))
    smat = pl.BlockSpec((H, H), lambda i: (0, 0))
    svec = pl.BlockSpec((1, H), lambda i: (0, 0))
    return pl.pallas_call(
        _gru_body,
        grid=(pl.cdiv(N, _NT),),
        in_specs=[nmat, nmat, nmat, nmat, nmat, smat, svec,
                  smat, smat, smat, smat, smat, smat,
                  svec, svec, svec, svec],
        out_specs=nmat,
        out_shape=jax.ShapeDtypeStruct((N, H), jnp.float32),
    )(h, a0, a1, d0, d1, rwt, cb, wir, wiz, win, whr, whz, whn,
      br, bz, bin_, bhn)


def _s2s_body(h_ref, a0_ref, a1_ref, d0_ref, d1_ref, rwt_ref, cb_ref,
              wir_ref, wiz_ref, win_ref, whr_ref, whz_ref, whn_ref,
              br_ref, bz_ref, bin_ref, bhn_ref, bt_ref,
              wii_ref, wif_ref, wig_ref, wio_ref,
              whi_ref, whf_ref, whg_ref, who_ref,
              bi_ref, bf_ref, bg_ref, bo_ref,
              rw1_ref, rb1_ref, rw2_ref, rb2_ref, nr_out_ref, out_ref):
    h = h_ref[...]
    deg = jnp.maximum(d0_ref[...] + d1_ref[...], 1.0)
    agg = (a0_ref[...] + a1_ref[...]) / deg
    m = jnp.dot(h, rwt_ref[...], preferred_element_type=jnp.float32) \
        + agg + cb_ref[...]
    r = jax.nn.sigmoid(
        jnp.dot(m, wir_ref[...], preferred_element_type=jnp.float32)
        + jnp.dot(h, whr_ref[...], preferred_element_type=jnp.float32)
        + br_ref[...])
    z = jax.nn.sigmoid(
        jnp.dot(m, wiz_ref[...], preferred_element_type=jnp.float32)
        + jnp.dot(h, whz_ref[...], preferred_element_type=jnp.float32)
        + bz_ref[...])
    nn = jnp.tanh(
        jnp.dot(m, win_ref[...], preferred_element_type=jnp.float32)
        + bin_ref[...]
        + r * (jnp.dot(h, whn_ref[...], preferred_element_type=jnp.float32)
               + bhn_ref[...]))
    nr = (1.0 - z) * nn + z * h
    nr_out_ref[...] = nr
    bt = bt_ref[...]
    gids = lax.broadcasted_iota(jnp.int32, (NG, 1), 0)
    maskb = bt == gids
    maskf = maskb.astype(jnp.float32)

    q_star = jnp.zeros((NG, 2 * H), jnp.float32)
    hh = jnp.zeros((NG, H), jnp.float32)
    cc = jnp.zeros((NG, H), jnp.float32)
    for _ in range(STEPS):
        def gate(wi, wh, b):
            return (jnp.dot(q_star, wi, preferred_element_type=jnp.float32)
                    + jnp.dot(hh, wh, preferred_element_type=jnp.float32)
                    + b)
        ig = jax.nn.sigmoid(gate(wii_ref[...], whi_ref[...], bi_ref[...]))
        fg = jax.nn.sigmoid(gate(wif_ref[...], whf_ref[...], bf_ref[...]))
        gg = jnp.tanh(gate(wig_ref[...], whg_ref[...], bg_ref[...]))
        og = jax.nn.sigmoid(gate(wio_ref[...], who_ref[...], bo_ref[...]))
        cc = fg * cc + ig * gg
        hh = og * jnp.tanh(cc)
        q = hh
        d = lax.dot_general(q, nr, (((1,), (1,)), ((), ())),
                            preferred_element_type=jnp.float32)
        neg = jnp.float32(-1e30)
        emax = jnp.max(jnp.where(maskb, d, neg), axis=1, keepdims=True)
        emax = jnp.where(emax > jnp.float32(-1e29), emax, 0.0)
        ex = jnp.where(maskb, jnp.exp(d - emax), 0.0)
        denom = jnp.sum(ex, axis=1, keepdims=True)
        a = ex / jnp.maximum(denom, jnp.float32(1e-30))
        r_read = jnp.dot(a, nr, preferred_element_type=jnp.float32)
        q_star = jnp.concatenate([q, r_read], axis=1)
    hid = jnp.maximum(
        jnp.dot(q_star, rw1_ref[...], preferred_element_type=jnp.float32)
        + rb1_ref[...], 0.0)
    out_ref[...] = jnp.dot(hid, rw2_ref[...],
                           preferred_element_type=jnp.float32) + rb2_ref[...]


def _tc_set2set(h, a0, a1, d0, d1, gru_args, bt2d,
                lstm_wih, lstm_whh, lstm_bih, lstm_bhh,
                ro_w1, ro_b1, ro_w2, ro_b2):
    wis = [lstm_wih[k * H:(k + 1) * H].T for k in range(4)]
    whs = [lstm_whh[k * H:(k + 1) * H].T for k in range(4)]
    bs = [(lstm_bih[k * H:(k + 1) * H]
           + lstm_bhh[k * H:(k + 1) * H]).reshape(1, H) for k in range(4)]
    return pl.pallas_call(
        _s2s_body,
        out_shape=(jax.ShapeDtypeStruct((N, H), jnp.float32),
                   jax.ShapeDtypeStruct((NG, H), jnp.float32)),
    )(h, a0, a1, d0, d1, *gru_args, bt2d, *wis, *whs, *bs,
      ro_w1.T, ro_b1.reshape(1, H), ro_w2.T, ro_b2.reshape(1, H))


def kernel(x, edge_index, edge_attr, batch, W1, b1, W2, b2, root_w, conv_b,
           gru_wih, gru_whh, gru_bih, gru_bhh, lstm_wih, lstm_whh, lstm_bih,
           lstm_bhh, ro_w1, ro_b1, ro_w2, ro_b2):
    npad = EP - E
    src2d = jnp.pad(edge_index[0], (0, npad)).reshape(NCHUNK, EC)
    dst2d = jnp.pad(edge_index[1], (0, npad),
                    constant_values=TRASH).reshape(NCHUNK, EC)
    attr_pad = jnp.pad(edge_attr, ((0, npad), (0, 0)))
    zeros_np = jnp.zeros((NP_, H), jnp.float32)
    ones_rows = jnp.ones((EC, H), jnp.float32)

    h = jnp.pad(x, ((0, 0), (0, H - FEAT)))

    w1t = W1.T
    b1r = b1.reshape(1, EDGE_HID)
    w2t = W2.T.astype(jnp.bfloat16)
    b2r = b2.reshape(H, H)
    expc = jnp.asarray(_EXP, dtype=jnp.bfloat16)
    redc = jnp.asarray(_RED, dtype=jnp.bfloat16)

    rwt = root_w.T
    cb = conv_b.reshape(1, H)
    wir, wiz, win = (gru_wih[k * H:(k + 1) * H].T for k in range(3))
    whr, whz, whn = (gru_whh[k * H:(k + 1) * H].T for k in range(3))
    br = (gru_bih[:H] + gru_bhh[:H]).reshape(1, H)
    bz = (gru_bih[H:2 * H] + gru_bhh[H:2 * H]).reshape(1, H)
    bin_ = gru_bih[2 * H:].reshape(1, H)
    bhn = gru_bhh[2 * H:].reshape(1, H)

    d0 = d1 = None
    for layer in range(LAYERS):
        if layer == 0:
            hsrc, degp = _sc_gather_deg(h, src2d, dst2d, zeros_np, ones_rows)
            d0, d1 = degp[0, :N], degp[1, :N]
        else:
            hsrc = _sc_gather(h, src2d)
        msgs = _tc_msgs(attr_pad, hsrc, w1t, b1r, w2t, b2r, expc, redc)
        aggp = _sc_scatter(msgs, dst2d, zeros_np)
        if layer < LAYERS - 1:
            h = _tc_gru(h, aggp[0, :N], aggp[1, :N], d0, d1, rwt, cb,
                        wir, wiz, win, whr, whz, whn, br, bz, bin_, bhn)

    gru_args = (rwt, cb, wir, wiz, win, whr, whz, whn, br, bz, bin_, bhn)
    node_repr, graph_repr = _tc_set2set(
        h, aggp[0, :N], aggp[1, :N], d0, d1, gru_args, batch.reshape(1, N),
        lstm_wih, lstm_whh, lstm_bih, lstm_bhh, ro_w1, ro_b1, ro_w2, ro_b2)
    return node_repr, graph_repr
